# baseline (speedup 1.0000x reference)
import functools

import jax
import jax.numpy as jnp
from jax import lax
from jax.experimental import pallas as pl
from jax.experimental.pallas import tpu as pltpu
from jax.experimental.pallas import tpu_sc as plsc

T = 2048
D = 768
FF = 1536
E = 64
NE = 80
NEP = 128
ZE_BASE = E
CAP = 64
SLOTS = E * CAP
TBL_ZERO = SLOTS
TBL_X = SLOTS + CAP
TBL_ROWS = TBL_X + T
ROW_BLK = 256
SLOT_BLK = 512

NC, NS, L = 2, 16, 16
NW = NC * NS
TOK_W = T // NW
SLOT_W = SLOTS // NW
NDUMMY = 64
XPAD_ROWS = T + NDUMMY


def _router_body(x_ref, wr_ref, bias_ref, r_out_ref, ypart_ref, tid_ref,
                 wslot_ref):
    x = x_ref[...]
    wr = wr_ref[...]
    logits = lax.dot_general(x, wr, (((1,), (1,)), ((), ())),
                             preferred_element_type=jnp.float32)
    col = lax.broadcasted_iota(jnp.int32, (T, NEP), 1)
    real = col < NE
    l = jnp.where(real, logits, -1e30)
    m = jnp.max(l, axis=1, keepdims=True)
    p = jnp.exp(l - m)
    p = jnp.where(real, p, 0.0)
    scores = p / jnp.sum(p, axis=1, keepdims=True)
    biased = jnp.where(real, scores + bias_ref[...], -1e30)
    bm = jnp.max(biased, axis=1, keepdims=True)
    sel = jnp.min(jnp.where(biased >= bm, col, NEP), axis=1, keepdims=True)
    w_tok = jnp.sum(jnp.where(col == sel, scores, 0.0), axis=1, keepdims=True)
    is_zero = sel >= ZE_BASE
    valid = sel < E
    oh = jnp.where((col == sel) & valid, 1.0, 0.0)
    run = jnp.zeros((1, NEP), jnp.float32)
    pos_blocks = []
    r_i = lax.broadcasted_iota(jnp.int32, (ROW_BLK, ROW_BLK), 0)
    c_i = lax.broadcasted_iota(jnp.int32, (ROW_BLK, ROW_BLK), 1)
    tril = jnp.where(r_i > c_i, 1.0, 0.0)
    for b in range(T // ROW_BLK):
        ohb = oh[b * ROW_BLK:(b + 1) * ROW_BLK]
        pb = lax.dot_general(tril, ohb, (((1,), (0,)), ((), ())),
                             preferred_element_type=jnp.float32) + run
        pos_blocks.append(pb)
        run = run + jnp.sum(ohb, axis=0, keepdims=True)
    posfull = jnp.concatenate(pos_blocks, axis=0)
    pos = jnp.sum(posfull * oh, axis=1, keepdims=True).astype(jnp.int32)
    keep = valid & (pos < CAP)
    slot = sel * CAP + pos
    tok = lax.broadcasted_iota(jnp.int32, (T, 1), 0)
    r_out_ref[...] = jnp.where(keep, slot,
                               jnp.where(is_zero, TBL_X + tok, TBL_ZERO))
    ypart_ref[...] = x * jnp.where(is_zero, w_tok, 0.0)
    slot_eff = jnp.where(keep, slot, -1)
    tokf = tok.astype(jnp.float32)
    tid_rows, w_rows = [], []
    for b in range(SLOTS // SLOT_BLK):
        cols = lax.broadcasted_iota(jnp.int32, (T, SLOT_BLK), 1) + b * SLOT_BLK
        mb = (cols == slot_eff).astype(jnp.float32)
        cnt = jnp.sum(mb, axis=0, keepdims=True)
        tsum = jnp.sum(mb * tokf, axis=0, keepdims=True)
        wsum = jnp.sum(mb * w_tok, axis=0, keepdims=True)
        colrow = lax.broadcasted_iota(jnp.int32, (1, SLOT_BLK), 1)
        dummy = (T + (colrow & (NDUMMY - 1))).astype(jnp.float32)
        tid_rows.append(jnp.where(cnt > 0.0, tsum, dummy))
        w_rows.append(wsum)
    tid_ref[...] = jnp.concatenate(tid_rows, axis=0).astype(jnp.int32)
    wslot_ref[...] = jnp.concatenate(w_rows, axis=0)


def _router(x, wr_pad, bias_pad):
    return pl.pallas_call(
        _router_body,
        out_shape=(
            jax.ShapeDtypeStruct((T, 1), jnp.int32),
            jax.ShapeDtypeStruct((T, D), jnp.float32),
            jax.ShapeDtypeStruct((SLOTS // SLOT_BLK, SLOT_BLK), jnp.int32),
            jax.ShapeDtypeStruct((SLOTS // SLOT_BLK, SLOT_BLK), jnp.float32),
        ),
    )(x, wr_pad, bias_pad)


def _dispatch_body(tid_hbm, xpad_hbm, buf_hbm, mytid_v, rows_v, sem):
    wid = lax.axis_index("s") * NC + lax.axis_index("c")
    base = wid * SLOT_W
    pltpu.sync_copy(tid_hbm.at[pl.ds(base, SLOT_W)], mytid_v)
    pltpu.async_copy(xpad_hbm.at[mytid_v], rows_v, sem).wait()
    pltpu.sync_copy(rows_v, buf_hbm.at[pl.ds(base, SLOT_W)])


def _dispatch(tid, x_pad):
    mesh = plsc.VectorSubcoreMesh(core_axis_name="c", subcore_axis_name="s")
    return pl.kernel(
        _dispatch_body,
        mesh=mesh,
        out_type=jax.ShapeDtypeStruct((SLOTS, D), jnp.float32),
        scratch_types=[
            pltpu.VMEM((SLOT_W,), jnp.int32),
            pltpu.VMEM((SLOT_W, D), jnp.float32),
            pltpu.SemaphoreType.DMA,
        ],
        compiler_params=pltpu.CompilerParams(needs_layout_passes=False),
    )(tid, x_pad)


FSPLIT = 2
FH = FF // FSPLIT


def _mlp_body(buf_ref, wslot_ref, wg_ref, wu_ref, wd_ref, yp_ref, out_ref):
    i = pl.program_id(0)
    j = pl.program_id(1)

    @pl.when(i < E)
    def _():
        xb = buf_ref[...].astype(jnp.bfloat16)
        g = jnp.dot(xb, wg_ref[0].astype(jnp.bfloat16),
                    preferred_element_type=jnp.float32)
        u = jnp.dot(xb, wu_ref[0].astype(jnp.bfloat16),
                    preferred_element_type=jnp.float32)
        h = g * jax.nn.sigmoid(g) * u
        o = jnp.dot(h.astype(jnp.bfloat16), wd_ref[0].astype(jnp.bfloat16),
                    preferred_element_type=jnp.float32)
        o = o * wslot_ref[pl.ds(i * CAP, CAP), :]

        @pl.when(j == 0)
        def _():
            out_ref[...] = o

        @pl.when(j > 0)
        def _():
            out_ref[...] += o

    @pl.when((i >= E) & (j == 0))
    def _():

        @pl.when(i == E)
        def _():
            out_ref[...] = jnp.zeros((CAP, D), jnp.float32)

        @pl.when(i > E)
        def _():
            out_ref[...] = yp_ref[...]


def _expert_mlp(buf, wslot2d, w_gate, w_up, w_down, y_partial):
    nsteps = E + 1 + T // CAP
    ce = lambda i, j: (jnp.minimum(i, E - 1), 0)
    return pl.pallas_call(
        _mlp_body,
        grid=(nsteps, FSPLIT),
        in_specs=[
            pl.BlockSpec((CAP, D), ce),
            pl.BlockSpec((SLOTS, 1), lambda i, j: (0, 0)),
            pl.BlockSpec((1, D, FH), lambda i, j: (jnp.minimum(i, E - 1), 0, j)),
            pl.BlockSpec((1, D, FH), lambda i, j: (jnp.minimum(i, E - 1), 0, j)),
            pl.BlockSpec((1, FH, D), lambda i, j: (jnp.minimum(i, E - 1), j, 0)),
            pl.BlockSpec((CAP, D), lambda i, j: (jnp.maximum(i - (E + 1), 0), 0)),
        ],
        out_specs=pl.BlockSpec((CAP, D), lambda i, j: (i, 0)),
        out_shape=jax.ShapeDtypeStruct((TBL_ROWS, D), jnp.float32),
        compiler_params=pltpu.CompilerParams(
            dimension_semantics=("arbitrary", "arbitrary")),
    )(buf, wslot2d, w_gate, w_up, w_down, y_partial)


def _combine_body(rout_hbm, table_hbm, y_hbm, myr_v, rows_v, sem):
    wid = lax.axis_index("s") * NC + lax.axis_index("c")
    base = wid * TOK_W
    pltpu.sync_copy(rout_hbm.at[pl.ds(base, TOK_W)], myr_v)
    pltpu.async_copy(table_hbm.at[myr_v], rows_v, sem).wait()
    pltpu.sync_copy(rows_v, y_hbm.at[pl.ds(base, TOK_W)])


def _combine(r_out, table):
    mesh = plsc.VectorSubcoreMesh(core_axis_name="c", subcore_axis_name="s")
    return pl.kernel(
        _combine_body,
        mesh=mesh,
        out_type=jax.ShapeDtypeStruct((T, D), jnp.float32),
        scratch_types=[
            pltpu.VMEM((TOK_W,), jnp.int32),
            pltpu.VMEM((TOK_W, D), jnp.float32),
            pltpu.SemaphoreType.DMA,
        ],
        compiler_params=pltpu.CompilerParams(needs_layout_passes=False),
    )(r_out, table)


def kernel(hidden_states, w_router, e_score_correction_bias, w_gate, w_up,
           w_down):
    x = hidden_states.astype(jnp.float32)
    wr_pad = jnp.pad(w_router.astype(jnp.float32), ((0, NEP - NE), (0, 0)))
    bias_pad = jnp.pad(e_score_correction_bias.astype(jnp.float32),
                       (0, NEP - NE)).reshape(1, NEP)
    r_out, y_partial, tid, wslot = _router(x, wr_pad, bias_pad)
    x_pad = jnp.pad(x, ((0, XPAD_ROWS - T), (0, 0)))
    buf = _dispatch(tid.reshape(-1), x_pad)
    table = _expert_mlp(buf, wslot.reshape(SLOTS, 1), w_gate, w_up, w_down,
                        y_partial)
    return _combine(r_out.reshape(-1), table)


# speedup vs baseline: 1.3645x; 1.3645x over previous
import functools

import jax
import jax.numpy as jnp
from jax import lax
from jax.experimental import pallas as pl
from jax.experimental.pallas import tpu as pltpu
from jax.experimental.pallas import tpu_sc as plsc

T = 2048
D = 768
FF = 1536
E = 64
NE = 80
NEP = 128
ZE_BASE = E
CAP = 64
SLOTS = E * CAP
TBL_ZERO = SLOTS
TBL_X = SLOTS + CAP
TBL_ROWS = TBL_X + T
ROW_BLK = 256

NC, NS, L = 2, 16, 16
NW = NC * NS
TOK_W = T // NW
SLOT_W = SLOTS // NW
NDUMMY = 64
XPAD_ROWS = T + NDUMMY


def _router_body(x_ref, wr_ref, bias_ref, r_out_ref, ypart_ref, tid_ref,
                 wslot_ref):
    x = x_ref[...]
    wr = wr_ref[...]
    logits = lax.dot_general(x, wr, (((1,), (1,)), ((), ())),
                             preferred_element_type=jnp.float32)
    col = lax.broadcasted_iota(jnp.int32, (T, NEP), 1)
    real = col < NE
    l = jnp.where(real, logits, -1e30)
    m = jnp.max(l, axis=1, keepdims=True)
    p = jnp.exp(l - m)
    p = jnp.where(real, p, 0.0)
    scores = p / jnp.sum(p, axis=1, keepdims=True)
    biased = jnp.where(real, scores + bias_ref[...], -1e30)
    bm = jnp.max(biased, axis=1, keepdims=True)
    sel = jnp.min(jnp.where(biased >= bm, col, NEP), axis=1, keepdims=True)
    w_tok = jnp.sum(jnp.where(col == sel, scores, 0.0), axis=1, keepdims=True)
    is_zero = sel >= ZE_BASE
    valid = sel < E
    oh = jnp.where((col == sel) & valid, 1.0, 0.0)
    run = jnp.zeros((1, NEP), jnp.float32)
    pos_blocks = []
    r_i = lax.broadcasted_iota(jnp.int32, (ROW_BLK, ROW_BLK), 0)
    c_i = lax.broadcasted_iota(jnp.int32, (ROW_BLK, ROW_BLK), 1)
    tril = jnp.where(r_i > c_i, 1.0, 0.0)
    for b in range(T // ROW_BLK):
        ohb = oh[b * ROW_BLK:(b + 1) * ROW_BLK]
        pb = lax.dot_general(tril, ohb, (((1,), (0,)), ((), ())),
                             preferred_element_type=jnp.float32) + run
        pos_blocks.append(pb)
        run = run + jnp.sum(ohb, axis=0, keepdims=True)
    posfull = jnp.concatenate(pos_blocks, axis=0)
    pos = jnp.sum(posfull * oh, axis=1, keepdims=True).astype(jnp.int32)
    keep = valid & (pos < CAP)
    slot = sel * CAP + pos
    tok = lax.broadcasted_iota(jnp.int32, (T, 1), 0)
    r_out_ref[...] = jnp.where(keep, slot,
                               jnp.where(is_zero, TBL_X + tok, TBL_ZERO))
    ypart_ref[...] = x * jnp.where(is_zero, w_tok, 0.0)
    tokf = tok.astype(jnp.float32)
    colp = lax.broadcasted_iota(jnp.int32, (T, CAP), 1)
    pmask = jnp.where((colp == pos) & keep, 1.0, 0.0)
    rhs = jnp.concatenate([pmask * tokf, pmask * w_tok, pmask],
                          axis=1)
    inv = lax.dot_general(oh, rhs, (((0,), (0,)), ((), ())),
                          preferred_element_type=jnp.float32)
    tsum = inv[:E, 0:CAP]
    wsum = inv[:E, CAP:2 * CAP]
    cnt = inv[:E, 2 * CAP:3 * CAP]
    dummy = (T + lax.broadcasted_iota(jnp.int32, (E, CAP), 1)).astype(
        jnp.float32)
    tid_ref[...] = jnp.round(jnp.where(cnt > 0.5, tsum, dummy)).astype(
        jnp.int32)
    wslot_ref[...] = wsum


def _router(x, wr_pad, bias_pad):
    return pl.pallas_call(
        _router_body,
        out_shape=(
            jax.ShapeDtypeStruct((T, 1), jnp.int32),
            jax.ShapeDtypeStruct((T, D), jnp.float32),
            jax.ShapeDtypeStruct((E, CAP), jnp.int32),
            jax.ShapeDtypeStruct((E, CAP), jnp.float32),
        ),
    )(x, wr_pad, bias_pad)


def _dispatch_body(tid_hbm, xpad_hbm, buf_hbm, mytid_v, rows_v, sem):
    wid = lax.axis_index("s") * NC + lax.axis_index("c")
    base = wid * SLOT_W
    pltpu.sync_copy(tid_hbm.at[pl.ds(base, SLOT_W)], mytid_v)
    pltpu.async_copy(xpad_hbm.at[mytid_v], rows_v, sem).wait()
    pltpu.sync_copy(rows_v, buf_hbm.at[pl.ds(base, SLOT_W)])


def _dispatch(tid, x_pad):
    mesh = plsc.VectorSubcoreMesh(core_axis_name="c", subcore_axis_name="s")
    return pl.kernel(
        _dispatch_body,
        mesh=mesh,
        out_type=jax.ShapeDtypeStruct((SLOTS, D), jnp.float32),
        scratch_types=[
            pltpu.VMEM((SLOT_W,), jnp.int32),
            pltpu.VMEM((SLOT_W, D), jnp.float32),
            pltpu.SemaphoreType.DMA,
        ],
        compiler_params=pltpu.CompilerParams(needs_layout_passes=False),
    )(tid, x_pad)


FSPLIT = 1
FH = FF // FSPLIT


def _mlp_body(buf_ref, wslot_ref, wg_ref, wu_ref, wd_ref, yp_ref, out_ref):
    i = pl.program_id(0)
    j = pl.program_id(1)

    @pl.when(i < E)
    def _():
        xb = buf_ref[...].astype(jnp.bfloat16)
        g = jnp.dot(xb, wg_ref[0].astype(jnp.bfloat16),
                    preferred_element_type=jnp.float32)
        u = jnp.dot(xb, wu_ref[0].astype(jnp.bfloat16),
                    preferred_element_type=jnp.float32)
        h = g * jax.nn.sigmoid(g) * u
        o = jnp.dot(h.astype(jnp.bfloat16), wd_ref[0].astype(jnp.bfloat16),
                    preferred_element_type=jnp.float32)
        o = o * wslot_ref[pl.ds(i * CAP, CAP), :]

        @pl.when(j == 0)
        def _():
            out_ref[...] = o

        @pl.when(j > 0)
        def _():
            out_ref[...] += o

    @pl.when((i >= E) & (j == 0))
    def _():

        @pl.when(i == E)
        def _():
            out_ref[...] = jnp.zeros((CAP, D), jnp.float32)

        @pl.when(i > E)
        def _():
            out_ref[...] = yp_ref[...]


def _expert_mlp(buf, wslot2d, w_gate, w_up, w_down, y_partial):
    nsteps = E + 1 + T // CAP
    ce = lambda i, j: (jnp.minimum(i, E - 1), 0)
    return pl.pallas_call(
        _mlp_body,
        grid=(nsteps, FSPLIT),
        in_specs=[
            pl.BlockSpec((CAP, D), ce),
            pl.BlockSpec((SLOTS, 1), lambda i, j: (0, 0)),
            pl.BlockSpec((1, D, FH), lambda i, j: (jnp.minimum(i, E - 1), 0, j)),
            pl.BlockSpec((1, D, FH), lambda i, j: (jnp.minimum(i, E - 1), 0, j)),
            pl.BlockSpec((1, FH, D), lambda i, j: (jnp.minimum(i, E - 1), j, 0)),
            pl.BlockSpec((CAP, D), lambda i, j: (jnp.maximum(i - (E + 1), 0), 0)),
        ],
        out_specs=pl.BlockSpec((CAP, D), lambda i, j: (i, 0)),
        out_shape=jax.ShapeDtypeStruct((TBL_ROWS, D), jnp.float32),
        compiler_params=pltpu.CompilerParams(
            dimension_semantics=("arbitrary", "arbitrary")),
    )(buf, wslot2d, w_gate, w_up, w_down, y_partial)


def _combine_body(rout_hbm, table_hbm, y_hbm, myr_v, rows_v, sem):
    wid = lax.axis_index("s") * NC + lax.axis_index("c")
    base = wid * TOK_W
    pltpu.sync_copy(rout_hbm.at[pl.ds(base, TOK_W)], myr_v)
    pltpu.async_copy(table_hbm.at[myr_v], rows_v, sem).wait()
    pltpu.sync_copy(rows_v, y_hbm.at[pl.ds(base, TOK_W)])


def _combine(r_out, table):
    mesh = plsc.VectorSubcoreMesh(core_axis_name="c", subcore_axis_name="s")
    return pl.kernel(
        _combine_body,
        mesh=mesh,
        out_type=jax.ShapeDtypeStruct((T, D), jnp.float32),
        scratch_types=[
            pltpu.VMEM((TOK_W,), jnp.int32),
            pltpu.VMEM((TOK_W, D), jnp.float32),
            pltpu.SemaphoreType.DMA,
        ],
        compiler_params=pltpu.CompilerParams(needs_layout_passes=False),
    )(r_out, table)


def kernel(hidden_states, w_router, e_score_correction_bias, w_gate, w_up,
           w_down):
    x = hidden_states.astype(jnp.float32)
    wr_pad = jnp.pad(w_router.astype(jnp.float32), ((0, NEP - NE), (0, 0)))
    bias_pad = jnp.pad(e_score_correction_bias.astype(jnp.float32),
                       (0, NEP - NE)).reshape(1, NEP)
    r_out, y_partial, tid, wslot = _router(x, wr_pad, bias_pad)
    x_pad = jnp.pad(x, ((0, XPAD_ROWS - T), (0, 0)))
    buf = _dispatch(tid.reshape(-1), x_pad)
    table = _expert_mlp(buf, wslot.reshape(SLOTS, 1), w_gate, w_up, w_down,
                        y_partial)
    return _combine(r_out.reshape(-1), table)


# speedup vs baseline: 1.3686x; 1.0029x over previous
import functools

import jax
import jax.numpy as jnp
from jax import lax
from jax.experimental import pallas as pl
from jax.experimental.pallas import tpu as pltpu
from jax.experimental.pallas import tpu_sc as plsc

T = 2048
D = 768
FF = 1536
E = 64
NE = 80
NEP = 128
ZE_BASE = E
CAP = 64
SLOTS = E * CAP
TBL_ZERO = SLOTS
TBL_X = SLOTS + CAP
TBL_ROWS = TBL_X + T
ROW_BLK = 256

NC, NS, L = 2, 16, 16
NW = NC * NS
TOK_W = T // NW
SLOT_W = SLOTS // NW
NDUMMY = 64
XPAD_ROWS = T + NDUMMY


def _router_body(x_ref, wr_ref, bias_ref, r_out_ref, ypart_ref, tid_ref,
                 wslot_ref):
    x = x_ref[...]
    wr = wr_ref[...]
    logits = lax.dot_general(x, wr, (((1,), (1,)), ((), ())),
                             preferred_element_type=jnp.float32)
    col = lax.broadcasted_iota(jnp.int32, (T, NEP), 1)
    real = col < NE
    l = jnp.where(real, logits, -1e30)
    m = jnp.max(l, axis=1, keepdims=True)
    p = jnp.exp(l - m)
    p = jnp.where(real, p, 0.0)
    scores = p / jnp.sum(p, axis=1, keepdims=True)
    biased = jnp.where(real, scores + bias_ref[...], -1e30)
    bm = jnp.max(biased, axis=1, keepdims=True)
    sel = jnp.min(jnp.where(biased >= bm, col, NEP), axis=1, keepdims=True)
    w_tok = jnp.sum(jnp.where(col == sel, scores, 0.0), axis=1, keepdims=True)
    is_zero = sel >= ZE_BASE
    valid = sel < E
    oh = jnp.where((col == sel) & valid, 1.0, 0.0)
    run = jnp.zeros((1, NEP), jnp.float32)
    pos_blocks = []
    r_i = lax.broadcasted_iota(jnp.int32, (ROW_BLK, ROW_BLK), 0)
    c_i = lax.broadcasted_iota(jnp.int32, (ROW_BLK, ROW_BLK), 1)
    tril = jnp.where(r_i > c_i, 1.0, 0.0)
    for b in range(T // ROW_BLK):
        ohb = oh[b * ROW_BLK:(b + 1) * ROW_BLK]
        pb = lax.dot_general(tril, ohb, (((1,), (0,)), ((), ())),
                             preferred_element_type=jnp.float32) + run
        pos_blocks.append(pb)
        run = run + jnp.sum(ohb, axis=0, keepdims=True)
    posfull = jnp.concatenate(pos_blocks, axis=0)
    pos = jnp.sum(posfull * oh, axis=1, keepdims=True).astype(jnp.int32)
    keep = valid & (pos < CAP)
    slot = sel * CAP + pos
    tok = lax.broadcasted_iota(jnp.int32, (T, 1), 0)
    r_out_ref[...] = jnp.where(keep, slot,
                               jnp.where(is_zero, TBL_X + tok, TBL_ZERO))
    ypart_ref[...] = x * jnp.where(is_zero, w_tok, 0.0)
    tokf = tok.astype(jnp.float32)
    colp = lax.broadcasted_iota(jnp.int32, (T, CAP), 1)
    pmask = jnp.where((colp == pos) & keep, 1.0, 0.0)
    rhs = jnp.concatenate([pmask * tokf, pmask * w_tok, pmask],
                          axis=1)
    inv = lax.dot_general(oh, rhs, (((0,), (0,)), ((), ())),
                          precision=lax.Precision.HIGHEST,
                          preferred_element_type=jnp.float32)
    tsum = inv[:E, 0:CAP]
    wsum = inv[:E, CAP:2 * CAP]
    cnt = inv[:E, 2 * CAP:3 * CAP]
    dummy = (T + lax.broadcasted_iota(jnp.int32, (E, CAP), 1)).astype(
        jnp.float32)
    tid_ref[...] = jnp.round(jnp.where(cnt > 0.5, tsum, dummy)).astype(
        jnp.int32)
    wslot_ref[...] = wsum


def _router(x, wr_pad, bias_pad):
    return pl.pallas_call(
        _router_body,
        out_shape=(
            jax.ShapeDtypeStruct((T, 1), jnp.int32),
            jax.ShapeDtypeStruct((T, D), jnp.float32),
            jax.ShapeDtypeStruct((E, CAP), jnp.int32),
            jax.ShapeDtypeStruct((E, CAP), jnp.float32),
        ),
    )(x, wr_pad, bias_pad)


def _dispatch_body(tid_hbm, xpad_hbm, buf_hbm, mytid_v, rows_v, sem):
    wid = lax.axis_index("s") * NC + lax.axis_index("c")
    base = wid * SLOT_W
    pltpu.sync_copy(tid_hbm.at[pl.ds(base, SLOT_W)], mytid_v)
    pltpu.async_copy(xpad_hbm.at[mytid_v], rows_v, sem).wait()
    pltpu.sync_copy(rows_v, buf_hbm.at[pl.ds(base, SLOT_W)])


def _dispatch(tid, x_pad):
    mesh = plsc.VectorSubcoreMesh(core_axis_name="c", subcore_axis_name="s")
    return pl.kernel(
        _dispatch_body,
        mesh=mesh,
        out_type=jax.ShapeDtypeStruct((SLOTS, D), jnp.float32),
        scratch_types=[
            pltpu.VMEM((SLOT_W,), jnp.int32),
            pltpu.VMEM((SLOT_W, D), jnp.float32),
            pltpu.SemaphoreType.DMA,
        ],
        compiler_params=pltpu.CompilerParams(needs_layout_passes=False),
    )(tid, x_pad)


FSPLIT = 1
FH = FF // FSPLIT


def _mlp_body(buf_ref, wslot_ref, wg_ref, wu_ref, wd_ref, yp_ref, out_ref):
    i = pl.program_id(0)
    j = pl.program_id(1)

    @pl.when(i < E)
    def _():
        xb = buf_ref[...].astype(jnp.bfloat16)
        g = jnp.dot(xb, wg_ref[0].astype(jnp.bfloat16),
                    preferred_element_type=jnp.float32)
        u = jnp.dot(xb, wu_ref[0].astype(jnp.bfloat16),
                    preferred_element_type=jnp.float32)
        h = g * jax.nn.sigmoid(g) * u
        o = jnp.dot(h.astype(jnp.bfloat16), wd_ref[0].astype(jnp.bfloat16),
                    preferred_element_type=jnp.float32)
        o = o * wslot_ref[pl.ds(i * CAP, CAP), :]

        @pl.when(j == 0)
        def _():
            out_ref[...] = o

        @pl.when(j > 0)
        def _():
            out_ref[...] += o

    @pl.when((i >= E) & (j == 0))
    def _():

        @pl.when(i == E)
        def _():
            out_ref[...] = jnp.zeros((CAP, D), jnp.float32)

        @pl.when(i > E)
        def _():
            out_ref[...] = yp_ref[...]


def _expert_mlp(buf, wslot2d, w_gate, w_up, w_down, y_partial):
    nsteps = E + 1 + T // CAP
    ce = lambda i, j: (jnp.minimum(i, E - 1), 0)
    return pl.pallas_call(
        _mlp_body,
        grid=(nsteps, FSPLIT),
        in_specs=[
            pl.BlockSpec((CAP, D), ce),
            pl.BlockSpec((SLOTS, 1), lambda i, j: (0, 0)),
            pl.BlockSpec((1, D, FH), lambda i, j: (jnp.minimum(i, E - 1), 0, j)),
            pl.BlockSpec((1, D, FH), lambda i, j: (jnp.minimum(i, E - 1), 0, j)),
            pl.BlockSpec((1, FH, D), lambda i, j: (jnp.minimum(i, E - 1), j, 0)),
            pl.BlockSpec((CAP, D), lambda i, j: (jnp.maximum(i - (E + 1), 0), 0)),
        ],
        out_specs=pl.BlockSpec((CAP, D), lambda i, j: (i, 0)),
        out_shape=jax.ShapeDtypeStruct((TBL_ROWS, D), jnp.float32),
        compiler_params=pltpu.CompilerParams(
            dimension_semantics=("arbitrary", "arbitrary")),
    )(buf, wslot2d, w_gate, w_up, w_down, y_partial)


def _combine_body(rout_hbm, table_hbm, y_hbm, myr_v, rows_v, sem):
    wid = lax.axis_index("s") * NC + lax.axis_index("c")
    base = wid * TOK_W
    pltpu.sync_copy(rout_hbm.at[pl.ds(base, TOK_W)], myr_v)
    pltpu.async_copy(table_hbm.at[myr_v], rows_v, sem).wait()
    pltpu.sync_copy(rows_v, y_hbm.at[pl.ds(base, TOK_W)])


def _combine(r_out, table):
    mesh = plsc.VectorSubcoreMesh(core_axis_name="c", subcore_axis_name="s")
    return pl.kernel(
        _combine_body,
        mesh=mesh,
        out_type=jax.ShapeDtypeStruct((T, D), jnp.float32),
        scratch_types=[
            pltpu.VMEM((TOK_W,), jnp.int32),
            pltpu.VMEM((TOK_W, D), jnp.float32),
            pltpu.SemaphoreType.DMA,
        ],
        compiler_params=pltpu.CompilerParams(needs_layout_passes=False),
    )(r_out, table)


def kernel(hidden_states, w_router, e_score_correction_bias, w_gate, w_up,
           w_down):
    x = hidden_states.astype(jnp.float32)
    wr_pad = jnp.pad(w_router.astype(jnp.float32), ((0, NEP - NE), (0, 0)))
    bias_pad = jnp.pad(e_score_correction_bias.astype(jnp.float32),
                       (0, NEP - NE)).reshape(1, NEP)
    r_out, y_partial, tid, wslot = _router(x, wr_pad, bias_pad)
    x_pad = jnp.pad(x, ((0, XPAD_ROWS - T), (0, 0)))
    buf = _dispatch(tid.reshape(-1), x_pad)
    table = _expert_mlp(buf, wslot.reshape(SLOTS, 1), w_gate, w_up, w_down,
                        y_partial)
    return _combine(r_out.reshape(-1), table)


# speedup vs baseline: 1.4054x; 1.0269x over previous
import functools

import jax
import jax.numpy as jnp
from jax import lax
from jax.experimental import pallas as pl
from jax.experimental.pallas import tpu as pltpu
from jax.experimental.pallas import tpu_sc as plsc

T = 2048
D = 768
FF = 1536
E = 64
NE = 80
NEP = 128
ZE_BASE = E
CAP = 64
SLOTS = E * CAP
TBL_ZERO = SLOTS
TBL_X = SLOTS + CAP
TBL_ROWS = TBL_X + T
ROW_BLK = 256

NC, NS, L = 2, 16, 16
NW = NC * NS
TOK_W = T // NW
SLOT_W = SLOTS // NW


def _router_body(x_ref, wr_ref, bias_ref, r_out_ref, ypart_ref, tid_ref,
                 wslot_ref):
    x = x_ref[...]
    wr = wr_ref[...]
    logits = lax.dot_general(x, wr, (((1,), (1,)), ((), ())),
                             preferred_element_type=jnp.float32)
    col = lax.broadcasted_iota(jnp.int32, (T, NEP), 1)
    real = col < NE
    l = jnp.where(real, logits, -1e30)
    m = jnp.max(l, axis=1, keepdims=True)
    p = jnp.exp(l - m)
    p = jnp.where(real, p, 0.0)
    scores = p / jnp.sum(p, axis=1, keepdims=True)
    biased = jnp.where(real, scores + bias_ref[...], -1e30)
    bm = jnp.max(biased, axis=1, keepdims=True)
    sel = jnp.min(jnp.where(biased >= bm, col, NEP), axis=1, keepdims=True)
    w_tok = jnp.sum(jnp.where(col == sel, scores, 0.0), axis=1, keepdims=True)
    is_zero = sel >= ZE_BASE
    valid = sel < E
    oh = jnp.where((col == sel) & valid, 1.0, 0.0)
    run = jnp.zeros((1, NEP), jnp.float32)
    pos_blocks = []
    r_i = lax.broadcasted_iota(jnp.int32, (ROW_BLK, ROW_BLK), 0)
    c_i = lax.broadcasted_iota(jnp.int32, (ROW_BLK, ROW_BLK), 1)
    tril = jnp.where(r_i > c_i, 1.0, 0.0)
    for b in range(T // ROW_BLK):
        ohb = oh[b * ROW_BLK:(b + 1) * ROW_BLK]
        pb = lax.dot_general(tril, ohb, (((1,), (0,)), ((), ())),
                             preferred_element_type=jnp.float32) + run
        pos_blocks.append(pb)
        run = run + jnp.sum(ohb, axis=0, keepdims=True)
    posfull = jnp.concatenate(pos_blocks, axis=0)
    pos = jnp.sum(posfull * oh, axis=1, keepdims=True).astype(jnp.int32)
    keep = valid & (pos < CAP)
    slot = sel * CAP + pos
    tok = lax.broadcasted_iota(jnp.int32, (T, 1), 0)
    r_out_ref[...] = jnp.where(keep, slot,
                               jnp.where(is_zero, TBL_X + tok, TBL_ZERO))
    ypart_ref[...] = x * jnp.where(is_zero, w_tok, 0.0)
    tokf = tok.astype(jnp.float32)
    colp = lax.broadcasted_iota(jnp.int32, (T, CAP), 1)
    pmask = jnp.where((colp == pos) & keep, 1.0, 0.0)
    rhs = jnp.concatenate([pmask * tokf, pmask * w_tok, pmask],
                          axis=1)
    inv = lax.dot_general(oh, rhs, (((0,), (0,)), ((), ())),
                          precision=lax.Precision.HIGHEST,
                          preferred_element_type=jnp.float32)
    tsum = inv[:E, 0:CAP]
    wsum = inv[:E, CAP:2 * CAP]
    cnt = inv[:E, 2 * CAP:3 * CAP]
    slotidx = (lax.broadcasted_iota(jnp.int32, (E, CAP), 0) * CAP
               + lax.broadcasted_iota(jnp.int32, (E, CAP), 1))
    dummy = (slotidx & (T - 1)).astype(jnp.float32)
    tid_ref[...] = jnp.round(jnp.where(cnt > 0.5, tsum, dummy)).astype(
        jnp.int32)
    wslot_ref[...] = wsum


def _router(x, wr_pad, bias_pad):
    return pl.pallas_call(
        _router_body,
        out_shape=(
            jax.ShapeDtypeStruct((T, 1), jnp.int32),
            jax.ShapeDtypeStruct((T, D), jnp.float32),
            jax.ShapeDtypeStruct((E, CAP), jnp.int32),
            jax.ShapeDtypeStruct((E, CAP), jnp.float32),
        ),
    )(x, wr_pad, bias_pad)


def _dispatch_body(tid_hbm, x_hbm, buf_hbm, mytid_v, rows_v, sem):
    wid = lax.axis_index("s") * NC + lax.axis_index("c")
    base = wid * SLOT_W
    pltpu.sync_copy(tid_hbm.at[pl.ds(base, SLOT_W)], mytid_v)
    pltpu.async_copy(x_hbm.at[mytid_v], rows_v, sem).wait()
    pltpu.sync_copy(rows_v, buf_hbm.at[pl.ds(base, SLOT_W)])


def _dispatch(tid, x):
    mesh = plsc.VectorSubcoreMesh(core_axis_name="c", subcore_axis_name="s")
    return pl.kernel(
        _dispatch_body,
        mesh=mesh,
        out_type=jax.ShapeDtypeStruct((SLOTS, D), jnp.float32),
        scratch_types=[
            pltpu.VMEM((SLOT_W,), jnp.int32),
            pltpu.VMEM((SLOT_W, D), jnp.float32),
            pltpu.SemaphoreType.DMA,
        ],
        compiler_params=pltpu.CompilerParams(needs_layout_passes=False),
    )(tid, x)


FSPLIT = 1
FH = FF // FSPLIT


def _mlp_body(buf_ref, wslot_ref, wg_ref, wu_ref, wd_ref, yp_ref, out_ref):
    i = pl.program_id(0)
    j = pl.program_id(1)

    @pl.when(i < E)
    def _():
        xb = buf_ref[...].astype(jnp.bfloat16)
        g = jnp.dot(xb, wg_ref[0].astype(jnp.bfloat16),
                    preferred_element_type=jnp.float32)
        u = jnp.dot(xb, wu_ref[0].astype(jnp.bfloat16),
                    preferred_element_type=jnp.float32)
        h = g * jax.nn.sigmoid(g) * u
        o = jnp.dot(h.astype(jnp.bfloat16), wd_ref[0].astype(jnp.bfloat16),
                    preferred_element_type=jnp.float32)
        o = o * wslot_ref[pl.ds(i * CAP, CAP), :]

        @pl.when(j == 0)
        def _():
            out_ref[...] = o

        @pl.when(j > 0)
        def _():
            out_ref[...] += o

    @pl.when((i >= E) & (j == 0))
    def _():

        @pl.when(i == E)
        def _():
            out_ref[...] = jnp.zeros((CAP, D), jnp.float32)

        @pl.when(i > E)
        def _():
            out_ref[...] = yp_ref[...]


def _expert_mlp(buf, wslot2d, w_gate, w_up, w_down, y_partial):
    nsteps = E + 1 + T // CAP
    ce = lambda i, j: (jnp.minimum(i, E - 1), 0)
    return pl.pallas_call(
        _mlp_body,
        grid=(nsteps, FSPLIT),
        in_specs=[
            pl.BlockSpec((CAP, D), ce),
            pl.BlockSpec((SLOTS, 1), lambda i, j: (0, 0)),
            pl.BlockSpec((1, D, FH), lambda i, j: (jnp.minimum(i, E - 1), 0, j)),
            pl.BlockSpec((1, D, FH), lambda i, j: (jnp.minimum(i, E - 1), 0, j)),
            pl.BlockSpec((1, FH, D), lambda i, j: (jnp.minimum(i, E - 1), j, 0)),
            pl.BlockSpec((CAP, D), lambda i, j: (jnp.maximum(i - (E + 1), 0), 0)),
        ],
        out_specs=pl.BlockSpec((CAP, D), lambda i, j: (i, 0)),
        out_shape=jax.ShapeDtypeStruct((TBL_ROWS, D), jnp.float32),
        compiler_params=pltpu.CompilerParams(
            dimension_semantics=("arbitrary", "arbitrary")),
    )(buf, wslot2d, w_gate, w_up, w_down, y_partial)


def _combine_body(rout_hbm, table_hbm, y_hbm, myr_v, rows_v, sem):
    wid = lax.axis_index("s") * NC + lax.axis_index("c")
    base = wid * TOK_W
    pltpu.sync_copy(rout_hbm.at[pl.ds(base, TOK_W)], myr_v)
    pltpu.async_copy(table_hbm.at[myr_v], rows_v, sem).wait()
    pltpu.sync_copy(rows_v, y_hbm.at[pl.ds(base, TOK_W)])


def _combine(r_out, table):
    mesh = plsc.VectorSubcoreMesh(core_axis_name="c", subcore_axis_name="s")
    return pl.kernel(
        _combine_body,
        mesh=mesh,
        out_type=jax.ShapeDtypeStruct((T, D), jnp.float32),
        scratch_types=[
            pltpu.VMEM((TOK_W,), jnp.int32),
            pltpu.VMEM((TOK_W, D), jnp.float32),
            pltpu.SemaphoreType.DMA,
        ],
        compiler_params=pltpu.CompilerParams(needs_layout_passes=False),
    )(r_out, table)


def kernel(hidden_states, w_router, e_score_correction_bias, w_gate, w_up,
           w_down):
    x = hidden_states.astype(jnp.float32)
    wr_pad = jnp.pad(w_router.astype(jnp.float32), ((0, NEP - NE), (0, 0)))
    bias_pad = jnp.pad(e_score_correction_bias.astype(jnp.float32),
                       (0, NEP - NE)).reshape(1, NEP)
    r_out, y_partial, tid, wslot = _router(x, wr_pad, bias_pad)
    buf = _dispatch(tid.reshape(-1), x)
    table = _expert_mlp(buf, wslot.reshape(SLOTS, 1), w_gate, w_up, w_down,
                        y_partial)
    return _combine(r_out.reshape(-1), table)


# speedup vs baseline: 1.4060x; 1.0004x over previous
import functools

import jax
import jax.numpy as jnp
from jax import lax
from jax.experimental import pallas as pl
from jax.experimental.pallas import tpu as pltpu
from jax.experimental.pallas import tpu_sc as plsc

T = 2048
D = 768
FF = 1536
E = 64
NE = 80
NEP = 128
ZE_BASE = E
CAP = 64
SLOTS = E * CAP
EPB = 2
BLK = EPB * CAP
TBL_ZERO = SLOTS
TBL_X = SLOTS + BLK
TBL_ROWS = TBL_X + T
ROW_BLK = 256

NC, NS, L = 2, 16, 16
NW = NC * NS
TOK_W = T // NW
SLOT_W = SLOTS // NW


def _router_body(x_ref, wr_ref, bias_ref, r_out_ref, ypart_ref, tid_ref,
                 wslot_ref):
    x = x_ref[...]
    wr = wr_ref[...]
    logits = lax.dot_general(x, wr, (((1,), (1,)), ((), ())),
                             preferred_element_type=jnp.float32)
    col = lax.broadcasted_iota(jnp.int32, (T, NEP), 1)
    real = col < NE
    l = jnp.where(real, logits, -1e30)
    m = jnp.max(l, axis=1, keepdims=True)
    p = jnp.exp(l - m)
    p = jnp.where(real, p, 0.0)
    scores = p / jnp.sum(p, axis=1, keepdims=True)
    biased = jnp.where(real, scores + bias_ref[...], -1e30)
    bm = jnp.max(biased, axis=1, keepdims=True)
    sel = jnp.min(jnp.where(biased >= bm, col, NEP), axis=1, keepdims=True)
    w_tok = jnp.sum(jnp.where(col == sel, scores, 0.0), axis=1, keepdims=True)
    is_zero = sel >= ZE_BASE
    valid = sel < E
    oh = jnp.where((col == sel) & valid, 1.0, 0.0)
    run = jnp.zeros((1, NEP), jnp.float32)
    pos_blocks = []
    r_i = lax.broadcasted_iota(jnp.int32, (ROW_BLK, ROW_BLK), 0)
    c_i = lax.broadcasted_iota(jnp.int32, (ROW_BLK, ROW_BLK), 1)
    tril = jnp.where(r_i > c_i, 1.0, 0.0)
    for b in range(T // ROW_BLK):
        ohb = oh[b * ROW_BLK:(b + 1) * ROW_BLK]
        pb = lax.dot_general(tril, ohb, (((1,), (0,)), ((), ())),
                             preferred_element_type=jnp.float32) + run
        pos_blocks.append(pb)
        run = run + jnp.sum(ohb, axis=0, keepdims=True)
    posfull = jnp.concatenate(pos_blocks, axis=0)
    pos = jnp.sum(posfull * oh, axis=1, keepdims=True).astype(jnp.int32)
    keep = valid & (pos < CAP)
    slot = sel * CAP + pos
    tok = lax.broadcasted_iota(jnp.int32, (T, 1), 0)
    r_out_ref[...] = jnp.where(keep, slot,
                               jnp.where(is_zero, TBL_X + tok, TBL_ZERO))
    ypart_ref[...] = x * jnp.where(is_zero, w_tok, 0.0)
    tokf = tok.astype(jnp.float32)
    colp = lax.broadcasted_iota(jnp.int32, (T, CAP), 1)
    pmask = jnp.where((colp == pos) & keep, 1.0, 0.0)
    rhs = jnp.concatenate([pmask * tokf, pmask * w_tok, pmask],
                          axis=1)
    inv = lax.dot_general(oh, rhs, (((0,), (0,)), ((), ())),
                          precision=lax.Precision.HIGHEST,
                          preferred_element_type=jnp.float32)
    tsum = inv[:E, 0:CAP]
    wsum = inv[:E, CAP:2 * CAP]
    cnt = inv[:E, 2 * CAP:3 * CAP]
    slotidx = (lax.broadcasted_iota(jnp.int32, (E, CAP), 0) * CAP
               + lax.broadcasted_iota(jnp.int32, (E, CAP), 1))
    dummy = (slotidx & (T - 1)).astype(jnp.float32)
    tid_ref[...] = jnp.round(jnp.where(cnt > 0.5, tsum, dummy)).astype(
        jnp.int32)
    wslot_ref[...] = wsum


def _router(x, wr_pad, bias_pad):
    return pl.pallas_call(
        _router_body,
        out_shape=(
            jax.ShapeDtypeStruct((T, 1), jnp.int32),
            jax.ShapeDtypeStruct((T, D), jnp.float32),
            jax.ShapeDtypeStruct((E, CAP), jnp.int32),
            jax.ShapeDtypeStruct((E, CAP), jnp.float32),
        ),
    )(x, wr_pad, bias_pad)


def _dispatch_body(tid_hbm, x_hbm, buf_hbm, mytid_v, rows_v, sem):
    wid = lax.axis_index("s") * NC + lax.axis_index("c")
    base = wid * SLOT_W
    pltpu.sync_copy(tid_hbm.at[pl.ds(base, SLOT_W)], mytid_v)
    pltpu.async_copy(x_hbm.at[mytid_v], rows_v, sem).wait()
    pltpu.sync_copy(rows_v, buf_hbm.at[pl.ds(base, SLOT_W)])


def _dispatch(tid, x):
    mesh = plsc.VectorSubcoreMesh(core_axis_name="c", subcore_axis_name="s")
    return pl.kernel(
        _dispatch_body,
        mesh=mesh,
        out_type=jax.ShapeDtypeStruct((SLOTS, D), jnp.float32),
        scratch_types=[
            pltpu.VMEM((SLOT_W,), jnp.int32),
            pltpu.VMEM((SLOT_W, D), jnp.float32),
            pltpu.SemaphoreType.DMA,
        ],
        compiler_params=pltpu.CompilerParams(needs_layout_passes=False),
    )(tid, x)


NEXP = E // EPB


def _mlp_body(buf_ref, wslot_ref, wg_ref, wu_ref, wd_ref, yp_ref, out_ref):
    i = pl.program_id(0)

    @pl.when(i < NEXP)
    def _():
        for e in range(EPB):
            xb = buf_ref[e * CAP:(e + 1) * CAP].astype(jnp.bfloat16)
            g = jnp.dot(xb, wg_ref[e].astype(jnp.bfloat16),
                        preferred_element_type=jnp.float32)
            u = jnp.dot(xb, wu_ref[e].astype(jnp.bfloat16),
                        preferred_element_type=jnp.float32)
            h = g * jax.nn.sigmoid(g) * u
            o = jnp.dot(h.astype(jnp.bfloat16), wd_ref[e].astype(jnp.bfloat16),
                        preferred_element_type=jnp.float32)
            out_ref[e * CAP:(e + 1) * CAP] = (
                o * wslot_ref[pl.ds(i * BLK + e * CAP, CAP), :])

    @pl.when(i == NEXP)
    def _():
        out_ref[...] = jnp.zeros((BLK, D), jnp.float32)

    @pl.when(i > NEXP)
    def _():
        out_ref[...] = yp_ref[...]


def _expert_mlp(buf, wslot2d, w_gate, w_up, w_down, y_partial):
    nsteps = NEXP + 1 + T // BLK
    return pl.pallas_call(
        _mlp_body,
        grid=(nsteps,),
        in_specs=[
            pl.BlockSpec((BLK, D), lambda i: (jnp.minimum(i, NEXP - 1), 0)),
            pl.BlockSpec((SLOTS, 1), lambda i: (0, 0)),
            pl.BlockSpec((EPB, D, FF), lambda i: (jnp.minimum(i, NEXP - 1), 0, 0)),
            pl.BlockSpec((EPB, D, FF), lambda i: (jnp.minimum(i, NEXP - 1), 0, 0)),
            pl.BlockSpec((EPB, FF, D), lambda i: (jnp.minimum(i, NEXP - 1), 0, 0)),
            pl.BlockSpec((BLK, D), lambda i: (jnp.maximum(i - (NEXP + 1), 0), 0)),
        ],
        out_specs=pl.BlockSpec((BLK, D), lambda i: (i, 0)),
        out_shape=jax.ShapeDtypeStruct((TBL_ROWS, D), jnp.float32),
        compiler_params=pltpu.CompilerParams(
            dimension_semantics=("arbitrary",),
            vmem_limit_bytes=120 * 1024 * 1024),
    )(buf, wslot2d, w_gate, w_up, w_down, y_partial)


def _combine_body(rout_hbm, table_hbm, y_hbm, myr_v, rows_v, sem):
    wid = lax.axis_index("s") * NC + lax.axis_index("c")
    base = wid * TOK_W
    pltpu.sync_copy(rout_hbm.at[pl.ds(base, TOK_W)], myr_v)
    pltpu.async_copy(table_hbm.at[myr_v], rows_v, sem).wait()
    pltpu.sync_copy(rows_v, y_hbm.at[pl.ds(base, TOK_W)])


def _combine(r_out, table):
    mesh = plsc.VectorSubcoreMesh(core_axis_name="c", subcore_axis_name="s")
    return pl.kernel(
        _combine_body,
        mesh=mesh,
        out_type=jax.ShapeDtypeStruct((T, D), jnp.float32),
        scratch_types=[
            pltpu.VMEM((TOK_W,), jnp.int32),
            pltpu.VMEM((TOK_W, D), jnp.float32),
            pltpu.SemaphoreType.DMA,
        ],
        compiler_params=pltpu.CompilerParams(needs_layout_passes=False),
    )(r_out, table)


def kernel(hidden_states, w_router, e_score_correction_bias, w_gate, w_up,
           w_down):
    x = hidden_states.astype(jnp.float32)
    wr_pad = jnp.pad(w_router.astype(jnp.float32), ((0, NEP - NE), (0, 0)))
    bias_pad = jnp.pad(e_score_correction_bias.astype(jnp.float32),
                       (0, NEP - NE)).reshape(1, NEP)
    r_out, y_partial, tid, wslot = _router(x, wr_pad, bias_pad)
    buf = _dispatch(tid.reshape(-1), x)
    table = _expert_mlp(buf, wslot.reshape(SLOTS, 1), w_gate, w_up, w_down,
                        y_partial)
    return _combine(r_out.reshape(-1), table)


# speedup vs baseline: 1.4076x; 1.0011x over previous
import functools

import jax
import jax.numpy as jnp
from jax import lax
from jax.experimental import pallas as pl
from jax.experimental.pallas import tpu as pltpu
from jax.experimental.pallas import tpu_sc as plsc

T = 2048
D = 768
FF = 1536
E = 64
NE = 80
NEP = 128
ZE_BASE = E
CAP = 64
SLOTS = E * CAP
EPB = 2
BLK = EPB * CAP
TBL_ZERO = SLOTS
TBL_X = SLOTS + BLK
TBL_ROWS = TBL_X + T
ROW_BLK = 256

NC, NS, L = 2, 16, 16
NW = NC * NS
TOK_W = T // NW
SLOT_W = SLOTS // NW


def _router_body(x_ref, wr_ref, bias_ref, r_out_ref, ypart_ref, tid_ref,
                 wslot_ref):
    x = x_ref[...]
    wr = wr_ref[...]
    logits = lax.dot_general(x, wr, (((1,), (1,)), ((), ())),
                             preferred_element_type=jnp.float32)
    col = lax.broadcasted_iota(jnp.int32, (T, NEP), 1)
    real = col < NE
    l = jnp.where(real, logits, -1e30)
    m = jnp.max(l, axis=1, keepdims=True)
    p = jnp.exp(l - m)
    p = jnp.where(real, p, 0.0)
    scores = p / jnp.sum(p, axis=1, keepdims=True)
    biased = jnp.where(real, scores + bias_ref[...], -1e30)
    bm = jnp.max(biased, axis=1, keepdims=True)
    sel = jnp.min(jnp.where(biased >= bm, col, NEP), axis=1, keepdims=True)
    w_tok = jnp.sum(jnp.where(col == sel, scores, 0.0), axis=1, keepdims=True)
    is_zero = sel >= ZE_BASE
    valid = sel < E
    oh = jnp.where((col == sel) & valid, 1.0, 0.0)
    run = jnp.zeros((1, NEP), jnp.float32)
    pos_blocks = []
    r_i = lax.broadcasted_iota(jnp.int32, (ROW_BLK, ROW_BLK), 0)
    c_i = lax.broadcasted_iota(jnp.int32, (ROW_BLK, ROW_BLK), 1)
    tril = jnp.where(r_i > c_i, 1.0, 0.0)
    for b in range(T // ROW_BLK):
        ohb = oh[b * ROW_BLK:(b + 1) * ROW_BLK]
        pb = lax.dot_general(tril, ohb, (((1,), (0,)), ((), ())),
                             preferred_element_type=jnp.float32) + run
        pos_blocks.append(pb)
        run = run + jnp.sum(ohb, axis=0, keepdims=True)
    posfull = jnp.concatenate(pos_blocks, axis=0)
    pos = jnp.sum(posfull * oh, axis=1, keepdims=True).astype(jnp.int32)
    keep = valid & (pos < CAP)
    slot = sel * CAP + pos
    tok = lax.broadcasted_iota(jnp.int32, (T, 1), 0)
    r_out_ref[...] = jnp.where(keep, slot,
                               jnp.where(is_zero, TBL_X + tok, TBL_ZERO))
    ypart_ref[...] = x * jnp.where(is_zero, w_tok, 0.0)
    tokf = tok.astype(jnp.float32)
    colp = lax.broadcasted_iota(jnp.int32, (T, CAP), 1)
    pmask = jnp.where((colp == pos) & keep, 1.0, 0.0)
    rhs = jnp.concatenate([pmask * tokf, pmask * w_tok, pmask],
                          axis=1)
    inv = lax.dot_general(oh, rhs, (((0,), (0,)), ((), ())),
                          precision=lax.Precision.HIGHEST,
                          preferred_element_type=jnp.float32)
    tsum = inv[:E, 0:CAP]
    wsum = inv[:E, CAP:2 * CAP]
    cnt = inv[:E, 2 * CAP:3 * CAP]
    slotidx = (lax.broadcasted_iota(jnp.int32, (E, CAP), 0) * CAP
               + lax.broadcasted_iota(jnp.int32, (E, CAP), 1))
    dummy = (slotidx & (T - 1)).astype(jnp.float32)
    tid_ref[...] = jnp.round(jnp.where(cnt > 0.5, tsum, dummy)).astype(
        jnp.int32)
    wslot_ref[...] = wsum


def _router(x, wr_pad, bias_pad):
    return pl.pallas_call(
        _router_body,
        out_shape=(
            jax.ShapeDtypeStruct((T, 1), jnp.int32),
            jax.ShapeDtypeStruct((T, D), jnp.float32),
            jax.ShapeDtypeStruct((E, CAP), jnp.int32),
            jax.ShapeDtypeStruct((E, CAP), jnp.float32),
        ),
    )(x, wr_pad, bias_pad)


SLOT_H = SLOT_W // 2


def _dispatch_body(tid_hbm, x_hbm, buf_hbm, tid_a, tid_b, rows_a, rows_b,
                   sem_a, sem_b):
    wid = lax.axis_index("s") * NC + lax.axis_index("c")
    base = wid * SLOT_W
    pltpu.sync_copy(tid_hbm.at[pl.ds(base, SLOT_H)], tid_a)
    pltpu.sync_copy(tid_hbm.at[pl.ds(base + SLOT_H, SLOT_H)], tid_b)
    cp_a = pltpu.async_copy(x_hbm.at[tid_a], rows_a, sem_a)
    cp_b = pltpu.async_copy(x_hbm.at[tid_b], rows_b, sem_b)
    cp_a.wait()
    pltpu.sync_copy(rows_a, buf_hbm.at[pl.ds(base, SLOT_H)])
    cp_b.wait()
    pltpu.sync_copy(rows_b, buf_hbm.at[pl.ds(base + SLOT_H, SLOT_H)])


def _dispatch(tid, x):
    mesh = plsc.VectorSubcoreMesh(core_axis_name="c", subcore_axis_name="s")
    return pl.kernel(
        _dispatch_body,
        mesh=mesh,
        out_type=jax.ShapeDtypeStruct((SLOTS, D), jnp.float32),
        scratch_types=[
            pltpu.VMEM((SLOT_H,), jnp.int32),
            pltpu.VMEM((SLOT_H,), jnp.int32),
            pltpu.VMEM((SLOT_H, D), jnp.float32),
            pltpu.VMEM((SLOT_H, D), jnp.float32),
            pltpu.SemaphoreType.DMA,
            pltpu.SemaphoreType.DMA,
        ],
        compiler_params=pltpu.CompilerParams(needs_layout_passes=False),
    )(tid, x)


NEXP = E // EPB


def _mlp_body(buf_ref, wslot_ref, wg_ref, wu_ref, wd_ref, yp_ref, out_ref):
    i = pl.program_id(0)

    @pl.when(i < NEXP)
    def _():
        for e in range(EPB):
            xb = buf_ref[e * CAP:(e + 1) * CAP].astype(jnp.bfloat16)
            g = jnp.dot(xb, wg_ref[e].astype(jnp.bfloat16),
                        preferred_element_type=jnp.float32)
            u = jnp.dot(xb, wu_ref[e].astype(jnp.bfloat16),
                        preferred_element_type=jnp.float32)
            h = g * jax.nn.sigmoid(g) * u
            o = jnp.dot(h.astype(jnp.bfloat16), wd_ref[e].astype(jnp.bfloat16),
                        preferred_element_type=jnp.float32)
            out_ref[e * CAP:(e + 1) * CAP] = (
                o * wslot_ref[pl.ds(i * BLK + e * CAP, CAP), :])

    @pl.when(i == NEXP)
    def _():
        out_ref[...] = jnp.zeros((BLK, D), jnp.float32)

    @pl.when(i > NEXP)
    def _():
        out_ref[...] = yp_ref[...]


def _expert_mlp(buf, wslot2d, w_gate, w_up, w_down, y_partial):
    nsteps = NEXP + 1 + T // BLK
    return pl.pallas_call(
        _mlp_body,
        grid=(nsteps,),
        in_specs=[
            pl.BlockSpec((BLK, D), lambda i: (jnp.minimum(i, NEXP - 1), 0)),
            pl.BlockSpec((SLOTS, 1), lambda i: (0, 0)),
            pl.BlockSpec((EPB, D, FF), lambda i: (jnp.minimum(i, NEXP - 1), 0, 0)),
            pl.BlockSpec((EPB, D, FF), lambda i: (jnp.minimum(i, NEXP - 1), 0, 0)),
            pl.BlockSpec((EPB, FF, D), lambda i: (jnp.minimum(i, NEXP - 1), 0, 0)),
            pl.BlockSpec((BLK, D), lambda i: (jnp.maximum(i - (NEXP + 1), 0), 0)),
        ],
        out_specs=pl.BlockSpec((BLK, D), lambda i: (i, 0)),
        out_shape=jax.ShapeDtypeStruct((TBL_ROWS, D), jnp.float32),
        compiler_params=pltpu.CompilerParams(
            dimension_semantics=("arbitrary",),
            vmem_limit_bytes=120 * 1024 * 1024),
    )(buf, wslot2d, w_gate, w_up, w_down, y_partial)


def _combine_body(rout_hbm, table_hbm, y_hbm, myr_v, rows_v, sem):
    wid = lax.axis_index("s") * NC + lax.axis_index("c")
    base = wid * TOK_W
    pltpu.sync_copy(rout_hbm.at[pl.ds(base, TOK_W)], myr_v)
    pltpu.async_copy(table_hbm.at[myr_v], rows_v, sem).wait()
    pltpu.sync_copy(rows_v, y_hbm.at[pl.ds(base, TOK_W)])


def _combine(r_out, table):
    mesh = plsc.VectorSubcoreMesh(core_axis_name="c", subcore_axis_name="s")
    return pl.kernel(
        _combine_body,
        mesh=mesh,
        out_type=jax.ShapeDtypeStruct((T, D), jnp.float32),
        scratch_types=[
            pltpu.VMEM((TOK_W,), jnp.int32),
            pltpu.VMEM((TOK_W, D), jnp.float32),
            pltpu.SemaphoreType.DMA,
        ],
        compiler_params=pltpu.CompilerParams(needs_layout_passes=False),
    )(r_out, table)


def kernel(hidden_states, w_router, e_score_correction_bias, w_gate, w_up,
           w_down):
    x = hidden_states.astype(jnp.float32)
    wr_pad = jnp.pad(w_router.astype(jnp.float32), ((0, NEP - NE), (0, 0)))
    bias_pad = jnp.pad(e_score_correction_bias.astype(jnp.float32),
                       (0, NEP - NE)).reshape(1, NEP)
    r_out, y_partial, tid, wslot = _router(x, wr_pad, bias_pad)
    buf = _dispatch(tid.reshape(-1), x)
    table = _expert_mlp(buf, wslot.reshape(SLOTS, 1), w_gate, w_up, w_down,
                        y_partial)
    return _combine(r_out.reshape(-1), table)


# speedup vs baseline: 1.4130x; 1.0038x over previous
import functools

import jax
import jax.numpy as jnp
from jax import lax
from jax.experimental import pallas as pl
from jax.experimental.pallas import tpu as pltpu
from jax.experimental.pallas import tpu_sc as plsc

T = 2048
D = 768
FF = 1536
E = 64
NE = 80
NEP = 128
ZE_BASE = E
CAP = 64
SLOTS = E * CAP
EPB = 2
BLK = EPB * CAP
TBL_ZERO = SLOTS
TBL_X = SLOTS + BLK
TBL_ROWS = TBL_X + T
ROW_BLK = 256

NC, NS, L = 2, 16, 16
NW = NC * NS
TOK_W = T // NW
SLOT_W = SLOTS // NW


def _router_body(x_ref, wr_ref, bias_ref, r_out_ref, wz_ref, tid_ref,
                 wslot_ref):
    x = x_ref[...]
    wr = wr_ref[...]
    logits = lax.dot_general(x, wr, (((1,), (1,)), ((), ())),
                             preferred_element_type=jnp.float32)
    col = lax.broadcasted_iota(jnp.int32, (T, NEP), 1)
    real = col < NE
    l = jnp.where(real, logits, -1e30)
    m = jnp.max(l, axis=1, keepdims=True)
    p = jnp.exp(l - m)
    p = jnp.where(real, p, 0.0)
    scores = p / jnp.sum(p, axis=1, keepdims=True)
    biased = jnp.where(real, scores + bias_ref[...], -1e30)
    bm = jnp.max(biased, axis=1, keepdims=True)
    sel = jnp.min(jnp.where(biased >= bm, col, NEP), axis=1, keepdims=True)
    w_tok = jnp.sum(jnp.where(col == sel, scores, 0.0), axis=1, keepdims=True)
    is_zero = sel >= ZE_BASE
    valid = sel < E
    oh = jnp.where((col == sel) & valid, 1.0, 0.0)
    run = jnp.zeros((1, NEP), jnp.float32)
    pos_blocks = []
    r_i = lax.broadcasted_iota(jnp.int32, (ROW_BLK, ROW_BLK), 0)
    c_i = lax.broadcasted_iota(jnp.int32, (ROW_BLK, ROW_BLK), 1)
    tril = jnp.where(r_i > c_i, 1.0, 0.0)
    for b in range(T // ROW_BLK):
        ohb = oh[b * ROW_BLK:(b + 1) * ROW_BLK]
        pb = lax.dot_general(tril, ohb, (((1,), (0,)), ((), ())),
                             preferred_element_type=jnp.float32) + run
        pos_blocks.append(pb)
        run = run + jnp.sum(ohb, axis=0, keepdims=True)
    posfull = jnp.concatenate(pos_blocks, axis=0)
    pos = jnp.sum(posfull * oh, axis=1, keepdims=True).astype(jnp.int32)
    keep = valid & (pos < CAP)
    slot = sel * CAP + pos
    tok = lax.broadcasted_iota(jnp.int32, (T, 1), 0)
    r_out_ref[...] = jnp.where(keep, slot,
                               jnp.where(is_zero, TBL_X + tok, TBL_ZERO))
    wz_ref[...] = jnp.where(is_zero, w_tok, 0.0)
    tokf = tok.astype(jnp.float32)
    colp = lax.broadcasted_iota(jnp.int32, (T, CAP), 1)
    pmask = jnp.where((colp == pos) & keep, 1.0, 0.0)
    rhs = jnp.concatenate([pmask * tokf, pmask * w_tok, pmask],
                          axis=1)
    inv = lax.dot_general(oh, rhs, (((0,), (0,)), ((), ())),
                          precision=lax.Precision.HIGHEST,
                          preferred_element_type=jnp.float32)
    tsum = inv[:E, 0:CAP]
    wsum = inv[:E, CAP:2 * CAP]
    cnt = inv[:E, 2 * CAP:3 * CAP]
    slotidx = (lax.broadcasted_iota(jnp.int32, (E, CAP), 0) * CAP
               + lax.broadcasted_iota(jnp.int32, (E, CAP), 1))
    dummy = (slotidx & (T - 1)).astype(jnp.float32)
    tid_ref[...] = jnp.round(jnp.where(cnt > 0.5, tsum, dummy)).astype(
        jnp.int32)
    wslot_ref[...] = wsum


def _router(x, wr_pad, bias_pad):
    return pl.pallas_call(
        _router_body,
        out_shape=(
            jax.ShapeDtypeStruct((T, 1), jnp.int32),
            jax.ShapeDtypeStruct((T, 1), jnp.float32),
            jax.ShapeDtypeStruct((E, CAP), jnp.int32),
            jax.ShapeDtypeStruct((E, CAP), jnp.float32),
        ),
    )(x, wr_pad, bias_pad)


SLOT_H = SLOT_W // 2


def _dispatch_body(tid_hbm, x_hbm, buf_hbm, tid_a, tid_b, rows_a, rows_b,
                   sem_a, sem_b):
    wid = lax.axis_index("s") * NC + lax.axis_index("c")
    base = wid * SLOT_W
    pltpu.sync_copy(tid_hbm.at[pl.ds(base, SLOT_H)], tid_a)
    pltpu.sync_copy(tid_hbm.at[pl.ds(base + SLOT_H, SLOT_H)], tid_b)
    cp_a = pltpu.async_copy(x_hbm.at[tid_a], rows_a, sem_a)
    cp_b = pltpu.async_copy(x_hbm.at[tid_b], rows_b, sem_b)
    cp_a.wait()
    pltpu.sync_copy(rows_a, buf_hbm.at[pl.ds(base, SLOT_H)])
    cp_b.wait()
    pltpu.sync_copy(rows_b, buf_hbm.at[pl.ds(base + SLOT_H, SLOT_H)])


def _dispatch(tid, x):
    mesh = plsc.VectorSubcoreMesh(core_axis_name="c", subcore_axis_name="s")
    return pl.kernel(
        _dispatch_body,
        mesh=mesh,
        out_type=jax.ShapeDtypeStruct((SLOTS, D), jnp.float32),
        scratch_types=[
            pltpu.VMEM((SLOT_H,), jnp.int32),
            pltpu.VMEM((SLOT_H,), jnp.int32),
            pltpu.VMEM((SLOT_H, D), jnp.float32),
            pltpu.VMEM((SLOT_H, D), jnp.float32),
            pltpu.SemaphoreType.DMA,
            pltpu.SemaphoreType.DMA,
        ],
        compiler_params=pltpu.CompilerParams(needs_layout_passes=False),
    )(tid, x)


NEXP = E // EPB


def _mlp_body(buf_ref, wslot_ref, wg_ref, wu_ref, wd_ref, x_ref, wz_ref,
              out_ref):
    i = pl.program_id(0)

    @pl.when(i < NEXP)
    def _():
        for e in range(EPB):
            xb = buf_ref[e * CAP:(e + 1) * CAP].astype(jnp.bfloat16)
            g = jnp.dot(xb, wg_ref[e].astype(jnp.bfloat16),
                        preferred_element_type=jnp.float32)
            u = jnp.dot(xb, wu_ref[e].astype(jnp.bfloat16),
                        preferred_element_type=jnp.float32)
            h = g * jax.nn.sigmoid(g) * u
            o = jnp.dot(h.astype(jnp.bfloat16), wd_ref[e].astype(jnp.bfloat16),
                        preferred_element_type=jnp.float32)
            out_ref[e * CAP:(e + 1) * CAP] = (
                o * wslot_ref[pl.ds(i * BLK + e * CAP, CAP), :])

    @pl.when(i == NEXP)
    def _():
        out_ref[...] = jnp.zeros((BLK, D), jnp.float32)

    @pl.when(i > NEXP)
    def _():
        blk0 = jnp.maximum(i - (NEXP + 1), 0) * BLK
        out_ref[...] = x_ref[...] * wz_ref[pl.ds(blk0, BLK), :]


def _expert_mlp(buf, wslot2d, w_gate, w_up, w_down, x, wz):
    nsteps = NEXP + 1 + T // BLK
    return pl.pallas_call(
        _mlp_body,
        grid=(nsteps,),
        in_specs=[
            pl.BlockSpec((BLK, D), lambda i: (jnp.minimum(i, NEXP - 1), 0)),
            pl.BlockSpec((SLOTS, 1), lambda i: (0, 0)),
            pl.BlockSpec((EPB, D, FF), lambda i: (jnp.minimum(i, NEXP - 1), 0, 0)),
            pl.BlockSpec((EPB, D, FF), lambda i: (jnp.minimum(i, NEXP - 1), 0, 0)),
            pl.BlockSpec((EPB, FF, D), lambda i: (jnp.minimum(i, NEXP - 1), 0, 0)),
            pl.BlockSpec((BLK, D), lambda i: (jnp.maximum(i - (NEXP + 1), 0), 0)),
            pl.BlockSpec((T, 1), lambda i: (0, 0)),
        ],
        out_specs=pl.BlockSpec((BLK, D), lambda i: (i, 0)),
        out_shape=jax.ShapeDtypeStruct((TBL_ROWS, D), jnp.float32),
        compiler_params=pltpu.CompilerParams(
            dimension_semantics=("arbitrary",),
            vmem_limit_bytes=63 * 1024 * 1024),
    )(buf, wslot2d, w_gate, w_up, w_down, x, wz)


def _combine_body(rout_hbm, table_hbm, y_hbm, myr_v, rows_v, sem):
    wid = lax.axis_index("s") * NC + lax.axis_index("c")
    base = wid * TOK_W
    pltpu.sync_copy(rout_hbm.at[pl.ds(base, TOK_W)], myr_v)
    pltpu.async_copy(table_hbm.at[myr_v], rows_v, sem).wait()
    pltpu.sync_copy(rows_v, y_hbm.at[pl.ds(base, TOK_W)])


def _combine(r_out, table):
    mesh = plsc.VectorSubcoreMesh(core_axis_name="c", subcore_axis_name="s")
    return pl.kernel(
        _combine_body,
        mesh=mesh,
        out_type=jax.ShapeDtypeStruct((T, D), jnp.float32),
        scratch_types=[
            pltpu.VMEM((TOK_W,), jnp.int32),
            pltpu.VMEM((TOK_W, D), jnp.float32),
            pltpu.SemaphoreType.DMA,
        ],
        compiler_params=pltpu.CompilerParams(needs_layout_passes=False),
    )(r_out, table)


def kernel(hidden_states, w_router, e_score_correction_bias, w_gate, w_up,
           w_down):
    x = hidden_states.astype(jnp.float32)
    wr_pad = jnp.pad(w_router.astype(jnp.float32), ((0, NEP - NE), (0, 0)))
    bias_pad = jnp.pad(e_score_correction_bias.astype(jnp.float32),
                       (0, NEP - NE)).reshape(1, NEP)
    r_out, wz, tid, wslot = _router(x, wr_pad, bias_pad)
    buf = _dispatch(tid.reshape(-1), x)
    table = _expert_mlp(buf, wslot.reshape(SLOTS, 1), w_gate, w_up, w_down,
                        x, wz)
    return _combine(r_out.reshape(-1), table)
